# Initial kernel scaffold; baseline (speedup 1.0000x reference)
import jax
import jax.numpy as jnp
from jax.experimental import pallas as pl


def kernel(x, edge_index, W0, b0, W1, b1):
    raise NotImplementedError("write your pallas kernel here")


# speedup vs baseline: 7.5557x; 7.5557x over previous
import functools

import jax
import jax.numpy as jnp
from jax import lax
from jax.experimental import pallas as pl
from jax.experimental.pallas import tpu as pltpu
from jax.experimental.pallas import tpu_sc as plsc

N = 10000
E = 320000
D = 128
NC = 2
NS = 16
NW = NC * NS
EPW = E // NW
C = 80
NCHUNK = EPW // C
NBLK = N // C

_mesh = plsc.VectorSubcoreMesh(core_axis_name="c", subcore_axis_name="s")


@functools.partial(
    pl.kernel,
    out_type=jax.ShapeDtypeStruct((NC * N,), jnp.float32),
    mesh=_mesh,
    scratch_types=[
        pltpu.VMEM((NCHUNK, C), jnp.int32),
        pltpu.VMEM((C,), jnp.float32),
        pltpu.VMEM((2000,), jnp.float32),
        pltpu.VMEM_SHARED((N,), jnp.float32),
    ],
)
def _deg_kernel(src_hbm, out_hbm, idx_v, ones_v, stage_v, acc_sh):
    c = lax.axis_index("c")
    s = lax.axis_index("s")
    w = c * NS + s

    def zstore(i, carry):
        stage_v[pl.ds(i * 16, 16)] = jnp.zeros((16,), jnp.float32)
        return carry

    lax.fori_loop(0, 125, zstore, 0)

    @pl.when(s < 5)
    def _zero():
        pltpu.sync_copy(stage_v, acc_sh.at[pl.ds(s * 2000, 2000)])

    pltpu.sync_copy(src_hbm.at[w], idx_v)
    for i in range(C // 16):
        ones_v[pl.ds(i * 16, 16)] = jnp.ones((16,), jnp.float32)
    plsc.subcore_barrier()

    def body(j, carry):
        pltpu.sync_copy(ones_v, acc_sh.at[idx_v.at[j]], add=True)
        return carry

    lax.fori_loop(0, NCHUNK, body, 0)
    plsc.subcore_barrier()

    @pl.when(s < 5)
    def _dump():
        pltpu.sync_copy(acc_sh.at[pl.ds(s * 2000, 2000)], stage_v)
        pltpu.sync_copy(stage_v, out_hbm.at[pl.ds(c * N + s * 2000, 2000)])


@functools.partial(
    pl.kernel,
    out_type=jax.ShapeDtypeStruct((NC, N, D), jnp.float32),
    mesh=_mesh,
    scratch_types=[
        pltpu.VMEM((NCHUNK, C), jnp.int32),
        pltpu.VMEM((NCHUNK, C), jnp.int32),
        pltpu.VMEM((C, D), jnp.float32),
        pltpu.VMEM_SHARED((N, D), jnp.float32),
        pltpu.SemaphoreType.DMA,
    ],
)
def _msg_kernel(xn_hbm, src_hbm, dst_hbm, out_hbm,
                sidx_v, didx_v, rows_v, acc_sh, sem):
    c = lax.axis_index("c")
    s = lax.axis_index("s")
    w = c * NS + s

    def zstore(i, carry):
        for j in range(D // 16):
            rows_v[i, pl.ds(j * 16, 16)] = jnp.zeros((16,), jnp.float32)
        return carry

    lax.fori_loop(0, C, zstore, 0)
    for k in range(8):
        blk = s * 8 + k

        @pl.when(blk < NBLK)
        def _z():
            pltpu.sync_copy(rows_v, acc_sh.at[pl.ds(blk * C, C)])
    pltpu.sync_copy(src_hbm.at[w], sidx_v)
    pltpu.sync_copy(dst_hbm.at[w], didx_v)
    plsc.subcore_barrier()

    def body(j, carry):
        pltpu.async_copy(xn_hbm.at[sidx_v.at[j]], rows_v, sem).wait()
        pltpu.sync_copy(rows_v, acc_sh.at[didx_v.at[j]], add=True)
        return carry

    lax.fori_loop(0, NCHUNK, body, 0)
    plsc.subcore_barrier()

    for k in range(8):
        blk = s * 8 + k

        @pl.when(blk < NBLK)
        def _d():
            pltpu.sync_copy(acc_sh.at[pl.ds(blk * C, C)], rows_v)
            pltpu.sync_copy(rows_v, out_hbm.at[c, pl.ds(blk * C, C)])


RB = 1000
GRID = N // RB


def _norm_body(dp0_ref, dp1_ref, x_ref, xn_ref):
    deg = jnp.maximum(dp0_ref[0, 0, :] + dp1_ref[0, 0, :], 1.0)
    xn_ref[...] = x_ref[...] * (1.0 / deg)[:, None]


_DPSPEC = pl.BlockSpec((1, 1, RB), lambda i: (i, 0, 0))


def _normalize(dp0, dp1, x):
    return pl.pallas_call(
        _norm_body,
        grid=(GRID,),
        in_specs=[
            _DPSPEC,
            _DPSPEC,
            pl.BlockSpec((RB, D), lambda i: (i, 0)),
        ],
        out_specs=pl.BlockSpec((RB, D), lambda i: (i, 0)),
        out_shape=jax.ShapeDtypeStruct((N, D), jnp.float32),
    )(dp0, dp1, x)


def _layer0_body(dp0_ref, dp1_ref, x_ref, hp_ref, w0a_ref, w0b_ref, b0_ref,
                 x1_ref, xn1_ref):
    h = hp_ref[0] + hp_ref[1]
    z = (jnp.dot(x_ref[...], w0a_ref[...], preferred_element_type=jnp.float32)
         + jnp.dot(h, w0b_ref[...], preferred_element_type=jnp.float32)
         + b0_ref[...])
    x1 = jnp.maximum(z, 0.0)
    x1_ref[...] = x1
    deg = jnp.maximum(dp0_ref[0, 0, :] + dp1_ref[0, 0, :], 1.0)
    xn1_ref[...] = x1 * (1.0 / deg)[:, None]


def _layer0(dp0, dp1, x, hp, w0a, w0b, b0):
    return pl.pallas_call(
        _layer0_body,
        grid=(GRID,),
        in_specs=[
            _DPSPEC,
            _DPSPEC,
            pl.BlockSpec((RB, D), lambda i: (i, 0)),
            pl.BlockSpec((NC, RB, D), lambda i: (0, i, 0)),
            pl.BlockSpec((D, D), lambda i: (0, 0)),
            pl.BlockSpec((D, D), lambda i: (0, 0)),
            pl.BlockSpec((1, D), lambda i: (0, 0)),
        ],
        out_specs=[
            pl.BlockSpec((RB, D), lambda i: (i, 0)),
            pl.BlockSpec((RB, D), lambda i: (i, 0)),
        ],
        out_shape=[
            jax.ShapeDtypeStruct((N, D), jnp.float32),
            jax.ShapeDtypeStruct((N, D), jnp.float32),
        ],
    )(dp0, dp1, x, hp, w0a, w0b, b0)


def _layer1_body(x1_ref, hp_ref, w1a_ref, w1b_ref, b1_ref, out_ref):
    h = hp_ref[0] + hp_ref[1]
    out_ref[...] = (
        jnp.dot(x1_ref[...], w1a_ref[...], preferred_element_type=jnp.float32)
        + jnp.dot(h, w1b_ref[...], preferred_element_type=jnp.float32)
        + b1_ref[...])


def _layer1(x1, hp, w1a, w1b, b1):
    return pl.pallas_call(
        _layer1_body,
        grid=(GRID,),
        in_specs=[
            pl.BlockSpec((RB, D), lambda i: (i, 0)),
            pl.BlockSpec((NC, RB, D), lambda i: (0, i, 0)),
            pl.BlockSpec((D, D), lambda i: (0, 0)),
            pl.BlockSpec((D, D), lambda i: (0, 0)),
            pl.BlockSpec((1, D), lambda i: (0, 0)),
        ],
        out_specs=pl.BlockSpec((RB, D), lambda i: (i, 0)),
        out_shape=jax.ShapeDtypeStruct((N, D), jnp.float32),
    )(x1, hp, w1a, w1b, b1)


def kernel(x, edge_index, W0, b0, W1, b1):
    ei = edge_index.astype(jnp.int32)
    src = ei[0].reshape(NW, NCHUNK, C)
    dst = ei[1].reshape(NW, NCHUNK, C)
    w0a, w0b = W0[:D], W0[D:]
    w1a, w1b = W1[:D], W1[D:]
    b0r = b0.reshape(1, D)
    b1r = b1.reshape(1, D)

    dpflat = _deg_kernel(src)
    dp0 = dpflat[:N].reshape(GRID, 1, RB)
    dp1 = dpflat[N:].reshape(GRID, 1, RB)
    xn0 = _normalize(dp0, dp1, x)
    hp0 = _msg_kernel(xn0, src, dst)
    x1, xn1 = _layer0(dp0, dp1, x, hp0, w0a, w0b, b0r)
    hp1 = _msg_kernel(xn1, src, dst)
    out = _layer1(x1, hp1, w1a, w1b, b1r)
    return out


# speedup vs baseline: 11.5267x; 1.5256x over previous
import functools

import jax
import jax.numpy as jnp
from jax import lax
from jax.experimental import pallas as pl
from jax.experimental.pallas import tpu as pltpu
from jax.experimental.pallas import tpu_sc as plsc

N = 10000
E = 320000
D = 128
NC = 2
NS = 16
NW = NC * NS
EPW = E // NW
C = 80
NCHUNK = EPW // C
NBLK = N // C
HCH = 63
PADE = NW * 2 * HCH * C - E
PADR = 16

_mesh = plsc.VectorSubcoreMesh(core_axis_name="c", subcore_axis_name="s")


@functools.partial(
    pl.kernel,
    out_type=jax.ShapeDtypeStruct((NC * N,), jnp.float32),
    mesh=_mesh,
    scratch_types=[
        pltpu.VMEM((NCHUNK, C), jnp.int32),
        pltpu.VMEM((C,), jnp.float32),
        pltpu.VMEM((2000,), jnp.float32),
        pltpu.VMEM_SHARED((N,), jnp.float32),
    ],
)
def _deg_kernel(src_hbm, out_hbm, idx_v, ones_v, stage_v, acc_sh):
    c = lax.axis_index("c")
    s = lax.axis_index("s")
    w = c * NS + s

    def zstore(i, carry):
        stage_v[pl.ds(i * 16, 16)] = jnp.zeros((16,), jnp.float32)
        return carry

    lax.fori_loop(0, 125, zstore, 0)

    @pl.when(s < 5)
    def _zero():
        pltpu.sync_copy(stage_v, acc_sh.at[pl.ds(s * 2000, 2000)])

    pltpu.sync_copy(src_hbm.at[w], idx_v)
    for i in range(C // 16):
        ones_v[pl.ds(i * 16, 16)] = jnp.ones((16,), jnp.float32)
    plsc.subcore_barrier()

    def body(j, carry):
        pltpu.sync_copy(ones_v, acc_sh.at[idx_v.at[j]], add=True)
        return carry

    lax.fori_loop(0, NCHUNK, body, 0)
    plsc.subcore_barrier()

    @pl.when(s < 5)
    def _dump():
        pltpu.sync_copy(acc_sh.at[pl.ds(s * 2000, 2000)], stage_v)
        pltpu.sync_copy(stage_v, out_hbm.at[pl.ds(c * N + s * 2000, 2000)])


@functools.partial(
    pl.kernel,
    out_type=jax.ShapeDtypeStruct((NC, N, D), jnp.float32),
    mesh=_mesh,
    scratch_types=[
        pltpu.VMEM((HCH, C), jnp.int32),
        pltpu.VMEM((HCH, C), jnp.int32),
        pltpu.VMEM((C, D), jnp.float32),
        pltpu.VMEM((C, D), jnp.float32),
        pltpu.VMEM_SHARED((N + PADR, D), jnp.float32),
        pltpu.SemaphoreType.DMA,
        pltpu.SemaphoreType.DMA,
    ],
)
def _msg_kernel(xn_hbm, src_hbm, dst_hbm, out_hbm,
                sidx_v, didx_v, rows0_v, rows1_v, acc_sh, sem0, sem1):
    c = lax.axis_index("c")
    s = lax.axis_index("s")
    w = c * NS + s

    def zstore(i, carry):
        for j in range(D // 16):
            rows0_v[i, pl.ds(j * 16, 16)] = jnp.zeros((16,), jnp.float32)
        return carry

    lax.fori_loop(0, C, zstore, 0)
    for k in range(8):
        blk = s * 8 + k

        @pl.when(blk < NBLK)
        def _z():
            pltpu.sync_copy(rows0_v, acc_sh.at[pl.ds(blk * C, C)])
    plsc.subcore_barrier()

    for half in range(2):
        nch = HCH
        pltpu.sync_copy(src_hbm.at[w, half], sidx_v)
        pltpu.sync_copy(dst_hbm.at[w, half], didx_v)
        pltpu.async_copy(xn_hbm.at[sidx_v.at[0]], rows0_v, sem0)

        def body(t, carry, n=nch):
            j0 = 2 * t

            @pl.when(j0 + 1 < n)
            def _g1():
                pltpu.async_copy(xn_hbm.at[sidx_v.at[j0 + 1]], rows1_v, sem1)

            pltpu.make_async_copy(xn_hbm.at[sidx_v.at[j0]],
                                  rows0_v, sem0).wait()
            pltpu.sync_copy(rows0_v, acc_sh.at[didx_v.at[j0]], add=True)

            @pl.when(j0 + 2 < n)
            def _g2():
                pltpu.async_copy(xn_hbm.at[sidx_v.at[j0 + 2]], rows0_v, sem0)

            @pl.when(j0 + 1 < n)
            def _s1():
                pltpu.make_async_copy(xn_hbm.at[sidx_v.at[j0 + 1]],
                                      rows1_v, sem1).wait()
                pltpu.sync_copy(rows1_v, acc_sh.at[didx_v.at[j0 + 1]],
                                add=True)

            return carry

        lax.fori_loop(0, (nch + 1) // 2, body, 0)
    plsc.subcore_barrier()

    for k in range(8):
        blk = s * 8 + k

        @pl.when(blk < NBLK)
        def _d():
            pltpu.sync_copy(acc_sh.at[pl.ds(blk * C, C)], rows0_v)
            pltpu.sync_copy(rows0_v, out_hbm.at[c, pl.ds(blk * C, C)])


RB = 1000
GRID = N // RB


def _norm_body(dp0_ref, dp1_ref, x_ref, xn_ref):
    deg = jnp.maximum(dp0_ref[0, 0, :] + dp1_ref[0, 0, :], 1.0)
    xn_ref[...] = x_ref[...] * (1.0 / deg)[:, None]


_DPSPEC = pl.BlockSpec((1, 1, RB), lambda i: (i, 0, 0))


def _normalize(dp0, dp1, x):
    return pl.pallas_call(
        _norm_body,
        grid=(GRID,),
        in_specs=[
            _DPSPEC,
            _DPSPEC,
            pl.BlockSpec((RB, D), lambda i: (i, 0)),
        ],
        out_specs=pl.BlockSpec((RB, D), lambda i: (i, 0)),
        out_shape=jax.ShapeDtypeStruct((N, D), jnp.float32),
    )(dp0, dp1, x)


def _layer0_body(dp0_ref, dp1_ref, x_ref, hp_ref, w0a_ref, w0b_ref, b0_ref,
                 x1_ref, xn1_ref):
    h = hp_ref[0] + hp_ref[1]
    z = (jnp.dot(x_ref[...], w0a_ref[...], preferred_element_type=jnp.float32)
         + jnp.dot(h, w0b_ref[...], preferred_element_type=jnp.float32)
         + b0_ref[...])
    x1 = jnp.maximum(z, 0.0)
    x1_ref[...] = x1
    deg = jnp.maximum(dp0_ref[0, 0, :] + dp1_ref[0, 0, :], 1.0)
    xn1_ref[...] = x1 * (1.0 / deg)[:, None]


def _layer0(dp0, dp1, x, hp, w0a, w0b, b0):
    return pl.pallas_call(
        _layer0_body,
        grid=(GRID,),
        in_specs=[
            _DPSPEC,
            _DPSPEC,
            pl.BlockSpec((RB, D), lambda i: (i, 0)),
            pl.BlockSpec((NC, RB, D), lambda i: (0, i, 0)),
            pl.BlockSpec((D, D), lambda i: (0, 0)),
            pl.BlockSpec((D, D), lambda i: (0, 0)),
            pl.BlockSpec((1, D), lambda i: (0, 0)),
        ],
        out_specs=[
            pl.BlockSpec((RB, D), lambda i: (i, 0)),
            pl.BlockSpec((RB, D), lambda i: (i, 0)),
        ],
        out_shape=[
            jax.ShapeDtypeStruct((N, D), jnp.float32),
            jax.ShapeDtypeStruct((N, D), jnp.float32),
        ],
    )(dp0, dp1, x, hp, w0a, w0b, b0)


def _layer1_body(x1_ref, hp_ref, w1a_ref, w1b_ref, b1_ref, out_ref):
    h = hp_ref[0] + hp_ref[1]
    out_ref[...] = (
        jnp.dot(x1_ref[...], w1a_ref[...], preferred_element_type=jnp.float32)
        + jnp.dot(h, w1b_ref[...], preferred_element_type=jnp.float32)
        + b1_ref[...])


def _layer1(x1, hp, w1a, w1b, b1):
    return pl.pallas_call(
        _layer1_body,
        grid=(GRID,),
        in_specs=[
            pl.BlockSpec((RB, D), lambda i: (i, 0)),
            pl.BlockSpec((NC, RB, D), lambda i: (0, i, 0)),
            pl.BlockSpec((D, D), lambda i: (0, 0)),
            pl.BlockSpec((D, D), lambda i: (0, 0)),
            pl.BlockSpec((1, D), lambda i: (0, 0)),
        ],
        out_specs=pl.BlockSpec((RB, D), lambda i: (i, 0)),
        out_shape=jax.ShapeDtypeStruct((N, D), jnp.float32),
    )(x1, hp, w1a, w1b, b1)


def kernel(x, edge_index, W0, b0, W1, b1):
    ei = edge_index.astype(jnp.int32)
    src = ei[0].reshape(NW, NCHUNK, C)
    dst = ei[1].reshape(NW, NCHUNK, C)
    pad_i = jnp.arange(PADE, dtype=jnp.int32)
    src_pad = (pad_i * 97) % N
    dst_pad = N + (pad_i % PADR)
    srcm = jnp.concatenate([ei[0], src_pad]).reshape(NW, 2, HCH, C)
    dstm = jnp.concatenate([ei[1], dst_pad]).reshape(NW, 2, HCH, C)
    w0a, w0b = W0[:D], W0[D:]
    w1a, w1b = W1[:D], W1[D:]
    b0r = b0.reshape(1, D)
    b1r = b1.reshape(1, D)

    dpflat = _deg_kernel(src)
    dp0 = dpflat[:N].reshape(GRID, 1, RB)
    dp1 = dpflat[N:].reshape(GRID, 1, RB)
    xn0 = _normalize(dp0, dp1, x)
    hp0 = _msg_kernel(xn0, srcm, dstm)
    x1, xn1 = _layer0(dp0, dp1, x, hp0, w0a, w0b, b0r)
    hp1 = _msg_kernel(xn1, srcm, dstm)
    out = _layer1(x1, hp1, w1a, w1b, b1r)
    return out


# speedup vs baseline: 12.2450x; 1.0623x over previous
import functools

import jax
import jax.numpy as jnp
from jax import lax
from jax.experimental import pallas as pl
from jax.experimental.pallas import tpu as pltpu
from jax.experimental.pallas import tpu_sc as plsc

N = 10000
E = 320000
D = 128
NC = 2
NS = 16
NW = NC * NS
EPW = E // NW
C = 80
NCHUNK = EPW // C
DB = 80
NBLK = N // DB
CM = 96
HCH = 53
PADE = NW * 2 * HCH * CM - E
PADR = 16

_mesh = plsc.VectorSubcoreMesh(core_axis_name="c", subcore_axis_name="s")


@functools.partial(
    pl.kernel,
    out_type=jax.ShapeDtypeStruct((NC * N,), jnp.float32),
    mesh=_mesh,
    scratch_types=[
        pltpu.VMEM((NCHUNK, C), jnp.int32),
        pltpu.VMEM((C,), jnp.float32),
        pltpu.VMEM((2000,), jnp.float32),
        pltpu.VMEM_SHARED((N,), jnp.float32),
    ],
)
def _deg_kernel(src_hbm, out_hbm, idx_v, ones_v, stage_v, acc_sh):
    c = lax.axis_index("c")
    s = lax.axis_index("s")
    w = c * NS + s

    def zstore(i, carry):
        stage_v[pl.ds(i * 16, 16)] = jnp.zeros((16,), jnp.float32)
        return carry

    lax.fori_loop(0, 125, zstore, 0)

    @pl.when(s < 5)
    def _zero():
        pltpu.sync_copy(stage_v, acc_sh.at[pl.ds(s * 2000, 2000)])

    pltpu.sync_copy(src_hbm.at[w], idx_v)
    for i in range(C // 16):
        ones_v[pl.ds(i * 16, 16)] = jnp.ones((16,), jnp.float32)
    plsc.subcore_barrier()

    def body(j, carry):
        pltpu.sync_copy(ones_v, acc_sh.at[idx_v.at[j]], add=True)
        return carry

    lax.fori_loop(0, NCHUNK, body, 0)
    plsc.subcore_barrier()

    @pl.when(s < 5)
    def _dump():
        pltpu.sync_copy(acc_sh.at[pl.ds(s * 2000, 2000)], stage_v)
        pltpu.sync_copy(stage_v, out_hbm.at[pl.ds(c * N + s * 2000, 2000)])


@functools.partial(
    pl.kernel,
    out_type=jax.ShapeDtypeStruct((NC, N, D), jnp.float32),
    mesh=_mesh,
    scratch_types=[
        pltpu.VMEM((HCH, CM), jnp.int32),
        pltpu.VMEM((HCH, CM), jnp.int32),
        pltpu.VMEM((CM, D), jnp.float32),
        pltpu.VMEM((CM, D), jnp.float32),
        pltpu.VMEM_SHARED((N + PADR, D), jnp.float32),
        pltpu.SemaphoreType.DMA,
        pltpu.SemaphoreType.DMA,
    ],
)
def _msg_kernel(xn_hbm, src_hbm, dst_hbm, out_hbm,
                sidx_v, didx_v, rows0_v, rows1_v, acc_sh, sem0, sem1):
    c = lax.axis_index("c")
    s = lax.axis_index("s")
    w = c * NS + s

    def zstore(i, carry):
        for j in range(D // 16):
            rows0_v[i, pl.ds(j * 16, 16)] = jnp.zeros((16,), jnp.float32)
        return carry

    lax.fori_loop(0, DB, zstore, 0)
    for k in range(8):
        blk = s * 8 + k

        @pl.when(blk < NBLK)
        def _z():
            pltpu.async_copy(rows0_v.at[pl.ds(0, DB)],
                             acc_sh.at[pl.ds(blk * DB, DB)], sem0)
    pltpu.sync_copy(src_hbm.at[w, 0], sidx_v)
    pltpu.sync_copy(dst_hbm.at[w, 0], didx_v)
    for k in range(8):
        blk = s * 8 + k

        @pl.when(blk < NBLK)
        def _zw():
            pltpu.make_async_copy(rows0_v.at[pl.ds(0, DB)],
                                  acc_sh.at[pl.ds(blk * DB, DB)], sem0).wait()
    plsc.subcore_barrier()

    for half in range(2):
        nch = HCH
        if half == 1:
            pltpu.sync_copy(src_hbm.at[w, half], sidx_v)
            pltpu.sync_copy(dst_hbm.at[w, half], didx_v)
        pltpu.async_copy(xn_hbm.at[sidx_v.at[0]], rows0_v, sem0)

        def body(t, carry, n=nch):
            j0 = 2 * t

            @pl.when(j0 + 1 < n)
            def _g1():
                pltpu.async_copy(xn_hbm.at[sidx_v.at[j0 + 1]], rows1_v, sem1)

            pltpu.make_async_copy(xn_hbm.at[sidx_v.at[j0]],
                                  rows0_v, sem0).wait()
            pltpu.sync_copy(rows0_v, acc_sh.at[didx_v.at[j0]], add=True)

            @pl.when(j0 + 2 < n)
            def _g2():
                pltpu.async_copy(xn_hbm.at[sidx_v.at[j0 + 2]], rows0_v, sem0)

            @pl.when(j0 + 1 < n)
            def _s1():
                pltpu.make_async_copy(xn_hbm.at[sidx_v.at[j0 + 1]],
                                      rows1_v, sem1).wait()
                pltpu.sync_copy(rows1_v, acc_sh.at[didx_v.at[j0 + 1]],
                                add=True)

            return carry

        lax.fori_loop(0, (nch + 1) // 2, body, 0)
    plsc.subcore_barrier()

    for k in range(8):
        blk = s * 8 + k
        buf = rows0_v if k % 2 == 0 else rows1_v
        sem = sem0 if k % 2 == 0 else sem1
        if k >= 2:
            pblk = s * 8 + (k - 2)

            @pl.when(pblk < NBLK)
            def _dw():
                pltpu.make_async_copy(
                    buf.at[pl.ds(0, DB)],
                    out_hbm.at[c, pl.ds(pblk * DB, DB)], sem).wait()

        @pl.when(blk < NBLK)
        def _d():
            pltpu.sync_copy(acc_sh.at[pl.ds(blk * DB, DB)],
                            buf.at[pl.ds(0, DB)])
            pltpu.async_copy(buf.at[pl.ds(0, DB)],
                             out_hbm.at[c, pl.ds(blk * DB, DB)], sem)
    for k in (6, 7):
        blk = s * 8 + k
        buf = rows0_v if k % 2 == 0 else rows1_v
        sem = sem0 if k % 2 == 0 else sem1

        @pl.when(blk < NBLK)
        def _dw2():
            pltpu.make_async_copy(buf.at[pl.ds(0, DB)],
                                  out_hbm.at[c, pl.ds(blk * DB, DB)],
                                  sem).wait()


RB = 1000
GRID = N // RB


def _norm_body(dp0_ref, dp1_ref, x_ref, xn_ref):
    deg = jnp.maximum(dp0_ref[0, 0, :] + dp1_ref[0, 0, :], 1.0)
    xn_ref[...] = x_ref[...] * (1.0 / deg)[:, None]


_DPSPEC = pl.BlockSpec((1, 1, RB), lambda i: (i, 0, 0))


def _normalize(dp0, dp1, x):
    return pl.pallas_call(
        _norm_body,
        grid=(GRID,),
        in_specs=[
            _DPSPEC,
            _DPSPEC,
            pl.BlockSpec((RB, D), lambda i: (i, 0)),
        ],
        out_specs=pl.BlockSpec((RB, D), lambda i: (i, 0)),
        out_shape=jax.ShapeDtypeStruct((N, D), jnp.float32),
    )(dp0, dp1, x)


def _layer0_body(dp0_ref, dp1_ref, x_ref, hp_ref, w0a_ref, w0b_ref, b0_ref,
                 x1_ref, xn1_ref):
    h = hp_ref[0] + hp_ref[1]
    z = (jnp.dot(x_ref[...], w0a_ref[...], preferred_element_type=jnp.float32)
         + jnp.dot(h, w0b_ref[...], preferred_element_type=jnp.float32)
         + b0_ref[...])
    x1 = jnp.maximum(z, 0.0)
    x1_ref[...] = x1
    deg = jnp.maximum(dp0_ref[0, 0, :] + dp1_ref[0, 0, :], 1.0)
    xn1_ref[...] = x1 * (1.0 / deg)[:, None]


def _layer0(dp0, dp1, x, hp, w0a, w0b, b0):
    return pl.pallas_call(
        _layer0_body,
        grid=(GRID,),
        in_specs=[
            _DPSPEC,
            _DPSPEC,
            pl.BlockSpec((RB, D), lambda i: (i, 0)),
            pl.BlockSpec((NC, RB, D), lambda i: (0, i, 0)),
            pl.BlockSpec((D, D), lambda i: (0, 0)),
            pl.BlockSpec((D, D), lambda i: (0, 0)),
            pl.BlockSpec((1, D), lambda i: (0, 0)),
        ],
        out_specs=[
            pl.BlockSpec((RB, D), lambda i: (i, 0)),
            pl.BlockSpec((RB, D), lambda i: (i, 0)),
        ],
        out_shape=[
            jax.ShapeDtypeStruct((N, D), jnp.float32),
            jax.ShapeDtypeStruct((N, D), jnp.float32),
        ],
    )(dp0, dp1, x, hp, w0a, w0b, b0)


def _layer1_body(x1_ref, hp_ref, w1a_ref, w1b_ref, b1_ref, out_ref):
    h = hp_ref[0] + hp_ref[1]
    out_ref[...] = (
        jnp.dot(x1_ref[...], w1a_ref[...], preferred_element_type=jnp.float32)
        + jnp.dot(h, w1b_ref[...], preferred_element_type=jnp.float32)
        + b1_ref[...])


def _layer1(x1, hp, w1a, w1b, b1):
    return pl.pallas_call(
        _layer1_body,
        grid=(GRID,),
        in_specs=[
            pl.BlockSpec((RB, D), lambda i: (i, 0)),
            pl.BlockSpec((NC, RB, D), lambda i: (0, i, 0)),
            pl.BlockSpec((D, D), lambda i: (0, 0)),
            pl.BlockSpec((D, D), lambda i: (0, 0)),
            pl.BlockSpec((1, D), lambda i: (0, 0)),
        ],
        out_specs=pl.BlockSpec((RB, D), lambda i: (i, 0)),
        out_shape=jax.ShapeDtypeStruct((N, D), jnp.float32),
    )(x1, hp, w1a, w1b, b1)


def kernel(x, edge_index, W0, b0, W1, b1):
    ei = edge_index.astype(jnp.int32)
    src = ei[0].reshape(NW, NCHUNK, C)
    dst = ei[1].reshape(NW, NCHUNK, C)
    pad_i = jnp.arange(PADE, dtype=jnp.int32)
    src_pad = (pad_i * 97) % N
    dst_pad = N + (pad_i % PADR)
    srcm = jnp.concatenate([ei[0], src_pad]).reshape(NW, 2, HCH, CM)
    dstm = jnp.concatenate([ei[1], dst_pad]).reshape(NW, 2, HCH, CM)
    w0a, w0b = W0[:D], W0[D:]
    w1a, w1b = W1[:D], W1[D:]
    b0r = b0.reshape(1, D)
    b1r = b1.reshape(1, D)

    dpflat = _deg_kernel(src)
    dp0 = dpflat[:N].reshape(GRID, 1, RB)
    dp1 = dpflat[N:].reshape(GRID, 1, RB)
    xn0 = _normalize(dp0, dp1, x)
    hp0 = _msg_kernel(xn0, srcm, dstm)
    x1, xn1 = _layer0(dp0, dp1, x, hp0, w0a, w0b, b0r)
    hp1 = _msg_kernel(xn1, srcm, dstm)
    out = _layer1(x1, hp1, w1a, w1b, b1r)
    return out
